# baseline (speedup 1.0000x reference)
import functools

import jax
import jax.numpy as jnp
from jax import lax
from jax.experimental import pallas as pl
from jax.experimental.pallas import tpu as pltpu
from jax.experimental.pallas import tpu_sc as plsc

_INFO = plsc.get_sparse_core_info()
_NC, _NS, _L = _INFO.num_cores, _INFO.num_subcores, _INFO.num_lanes
_NW = _NC * _NS


def _make_lookup(n_total: int, n_values: int, chunk: int):
    per_w = n_total // _NW
    assert per_w * _NW == n_total
    assert per_w % chunk == 0
    n_chunks = per_w // chunk
    mesh = plsc.VectorSubcoreMesh(core_axis_name="c", subcore_axis_name="s")
    n_pad = 16 * ((n_values + 15) // 16)

    @functools.partial(
        pl.kernel,
        mesh=mesh,
        out_type=jax.ShapeDtypeStruct((n_total,), jnp.float32),
        scratch_types=[
            pltpu.VMEM((n_pad,), jnp.float32),
            pltpu.VMEM((chunk,), jnp.int32),
            pltpu.VMEM((chunk,), jnp.float32),
        ],
        compiler_params=pltpu.CompilerParams(needs_layout_passes=False),
    )
    def lookup(values_hbm, idx_hbm, out_hbm, table_v, idx_v, out_v):
        wid = lax.axis_index("s") * _NC + lax.axis_index("c")
        base = wid * per_w
        pltpu.sync_copy(values_hbm, table_v.at[pl.ds(0, n_values)])

        def do_chunk(g, carry):
            off = base + g * chunk
            pltpu.sync_copy(idx_hbm.at[pl.ds(off, chunk)], idx_v)

            def gather_step(i, c):
                iv = idx_v[pl.ds(i * _L, _L)]
                out_v[pl.ds(i * _L, _L)] = plsc.load_gather(table_v, [iv])
                return c

            lax.fori_loop(0, chunk // _L, gather_step, 0, unroll=8)
            pltpu.sync_copy(out_v, out_hbm.at[pl.ds(off, chunk)])
            return carry

        lax.fori_loop(0, n_chunks, do_chunk, 0)

    return lookup


def kernel(values, index):
    n_structure, n_atoms = index.shape
    n_total = n_structure * n_atoms
    lookup = _make_lookup(n_total, values.shape[0], chunk=12800)
    flat = lookup(values, index.reshape(-1))
    return flat.reshape(n_structure, n_atoms)


# speedup vs baseline: 1.4426x; 1.4426x over previous
import functools

import jax
import jax.numpy as jnp
from jax import lax
from jax.experimental import pallas as pl
from jax.experimental.pallas import tpu as pltpu
from jax.experimental.pallas import tpu_sc as plsc

_INFO = plsc.get_sparse_core_info()
_NC, _NS, _L = _INFO.num_cores, _INFO.num_subcores, _INFO.num_lanes
_NW = _NC * _NS
_NBUF = 2


def _make_lookup(n_total: int, n_values: int, chunk: int):
    per_w = n_total // _NW
    assert per_w * _NW == n_total
    assert per_w % chunk == 0 and chunk % _L == 0
    n_chunks = per_w // chunk
    assert n_chunks % _NBUF == 0
    mesh = plsc.VectorSubcoreMesh(core_axis_name="c", subcore_axis_name="s")
    n_pad = 16 * ((n_values + 15) // 16)

    @functools.partial(
        pl.kernel,
        mesh=mesh,
        out_type=jax.ShapeDtypeStruct((n_total,), jnp.float32),
        scratch_types=[
            pltpu.VMEM((n_pad,), jnp.float32),
            pltpu.VMEM((_NBUF, chunk), jnp.int32),
            pltpu.VMEM((_NBUF, chunk), jnp.float32),
        ]
        + [pltpu.SemaphoreType.DMA] * (2 * _NBUF),
        compiler_params=pltpu.CompilerParams(needs_layout_passes=False),
    )
    def lookup(values_hbm, idx_hbm, out_hbm, table_v, idx_v, out_v, *sems):
        in_sems, out_sems = sems[:_NBUF], sems[_NBUF:]
        wid = lax.axis_index("s") * _NC + lax.axis_index("c")
        base = wid * per_w
        pltpu.sync_copy(values_hbm, table_v.at[pl.ds(0, n_values)])

        for b in range(_NBUF):
            pltpu.async_copy(
                idx_hbm.at[pl.ds(base + b * chunk, chunk)], idx_v.at[b], in_sems[b]
            )

        @pl.loop(0, n_chunks // _NBUF)
        def _outer(o):
            for b in range(_NBUF):
                g = o * _NBUF + b
                off = base + g * chunk
                pltpu.make_async_copy(
                    idx_hbm.at[pl.ds(off, chunk)], idx_v.at[b], in_sems[b]
                ).wait()

                @pl.when(g >= _NBUF)
                def _():
                    pltpu.make_async_copy(
                        out_v.at[b], out_hbm.at[pl.ds(off, chunk)], out_sems[b]
                    ).wait()

                @plsc.parallel_loop(0, chunk // _L, unroll=8)
                def _gather(i):
                    iv = idx_v[b, pl.ds(i * _L, _L)]
                    out_v[b, pl.ds(i * _L, _L)] = plsc.load_gather(table_v, [iv])

                pltpu.async_copy(
                    out_v.at[b], out_hbm.at[pl.ds(off, chunk)], out_sems[b]
                )

                @pl.when(g + _NBUF < n_chunks)
                def _():
                    nxt = base + (g + _NBUF) * chunk
                    pltpu.async_copy(
                        idx_hbm.at[pl.ds(nxt, chunk)], idx_v.at[b], in_sems[b]
                    )

        for b in range(_NBUF):
            off = base + (n_chunks - _NBUF + b) * chunk
            pltpu.make_async_copy(
                out_v.at[b], out_hbm.at[pl.ds(off, chunk)], out_sems[b]
            ).wait()

    return lookup


def kernel(values, index):
    n_structure, n_atoms = index.shape
    n_total = n_structure * n_atoms
    lookup = _make_lookup(n_total, values.shape[0], chunk=6400)
    flat = lookup(values, index.reshape(-1))
    return flat.reshape(n_structure, n_atoms)


# speedup vs baseline: 2.6780x; 1.8564x over previous
import functools

import jax
import jax.numpy as jnp
from jax import lax
from jax.experimental import pallas as pl
from jax.experimental.pallas import tpu as pltpu
from jax.experimental.pallas import tpu_sc as plsc

_INFO = plsc.get_sparse_core_info()
_NC, _NS, _L = _INFO.num_cores, _INFO.num_subcores, _INFO.num_lanes
_NW = _NC * _NS
_NBUF = 2


def _make_lookup(n_rows: int, n_cols: int, n_values: int, chunk_rows: int):
    rows_per_w = n_rows // _NW
    assert rows_per_w * _NW == n_rows
    assert rows_per_w % chunk_rows == 0
    n_chunks = rows_per_w // chunk_rows
    assert n_chunks % _NBUF == 0
    col_offs = list(range(0, n_cols - _L + 1, _L))
    if col_offs[-1] != n_cols - _L:
        col_offs.append(n_cols - _L)
    mesh = plsc.VectorSubcoreMesh(core_axis_name="c", subcore_axis_name="s")
    n_pad = 16 * ((n_values + 15) // 16)

    @functools.partial(
        pl.kernel,
        mesh=mesh,
        out_type=jax.ShapeDtypeStruct((n_rows, n_cols), jnp.float32),
        scratch_types=[
            pltpu.VMEM((n_pad,), jnp.float32),
            pltpu.VMEM((_NBUF, chunk_rows, n_cols), jnp.int32),
            pltpu.VMEM((_NBUF, chunk_rows, n_cols), jnp.float32),
        ]
        + [pltpu.SemaphoreType.DMA] * (2 * _NBUF),
        compiler_params=pltpu.CompilerParams(needs_layout_passes=False),
    )
    def lookup(values_hbm, idx_hbm, out_hbm, table_v, idx_v, out_v, *sems):
        in_sems, out_sems = sems[:_NBUF], sems[_NBUF:]
        wid = lax.axis_index("s") * _NC + lax.axis_index("c")
        base = wid * rows_per_w
        pltpu.sync_copy(values_hbm, table_v.at[pl.ds(0, n_values)])

        for b in range(_NBUF):
            pltpu.async_copy(
                idx_hbm.at[pl.ds(base + b * chunk_rows, chunk_rows), :],
                idx_v.at[b],
                in_sems[b],
            )

        @pl.loop(0, n_chunks // _NBUF)
        def _outer(o):
            for b in range(_NBUF):
                g = o * _NBUF + b
                row0 = base + g * chunk_rows
                rows = pl.ds(row0, chunk_rows)
                pltpu.make_async_copy(
                    idx_hbm.at[rows, :], idx_v.at[b], in_sems[b]
                ).wait()

                @pl.when(g >= _NBUF)
                def _():
                    pltpu.make_async_copy(
                        out_v.at[b], out_hbm.at[rows, :], out_sems[b]
                    ).wait()

                @plsc.parallel_loop(0, chunk_rows, unroll=2)
                def _row(r):
                    for k in col_offs:
                        iv = idx_v[b, r, pl.ds(k, _L)]
                        out_v[b, r, pl.ds(k, _L)] = plsc.load_gather(
                            table_v, [iv]
                        )

                pltpu.async_copy(out_v.at[b], out_hbm.at[rows, :], out_sems[b])

                @pl.when(g + _NBUF < n_chunks)
                def _():
                    nxt = pl.ds(base + (g + _NBUF) * chunk_rows, chunk_rows)
                    pltpu.async_copy(idx_hbm.at[nxt, :], idx_v.at[b], in_sems[b])

        for b in range(_NBUF):
            rows = pl.ds(base + (n_chunks - _NBUF + b) * chunk_rows, chunk_rows)
            pltpu.make_async_copy(
                out_v.at[b], out_hbm.at[rows, :], out_sems[b]
            ).wait()

    return lookup


def kernel(values, index):
    n_rows, n_cols = index.shape
    lookup = _make_lookup(n_rows, n_cols, values.shape[0], chunk_rows=64)
    return lookup(values, index)


# speedup vs baseline: 2.6898x; 1.0044x over previous
import functools

import jax
import jax.numpy as jnp
from jax import lax
from jax.experimental import pallas as pl
from jax.experimental.pallas import tpu as pltpu
from jax.experimental.pallas import tpu_sc as plsc

_INFO = plsc.get_sparse_core_info()
_NC, _NS, _L = _INFO.num_cores, _INFO.num_subcores, _INFO.num_lanes
_NW = _NC * _NS
_NBUF = 2


def _make_lookup(n_rows: int, n_cols: int, n_values: int, chunk_rows: int):
    rows_per_w = n_rows // _NW
    assert rows_per_w * _NW == n_rows
    assert rows_per_w % chunk_rows == 0
    n_chunks = rows_per_w // chunk_rows
    assert n_chunks % _NBUF == 0
    col_offs = list(range(0, n_cols - _L + 1, _L))
    if col_offs[-1] != n_cols - _L:
        col_offs.append(n_cols - _L)
    mesh = plsc.VectorSubcoreMesh(core_axis_name="c", subcore_axis_name="s")
    n_pad = 16 * ((n_values + 15) // 16)

    @functools.partial(
        pl.kernel,
        mesh=mesh,
        out_type=jax.ShapeDtypeStruct((n_rows, n_cols), jnp.float32),
        scratch_types=[
            pltpu.VMEM((n_pad,), jnp.float32),
            pltpu.VMEM((_NBUF, chunk_rows, n_cols), jnp.int32),
            pltpu.VMEM((_NBUF, chunk_rows, n_cols), jnp.float32),
        ]
        + [pltpu.SemaphoreType.DMA] * (2 * _NBUF),
        compiler_params=pltpu.CompilerParams(
            needs_layout_passes=False, use_tc_tiling_on_sc=True
        ),
    )
    def lookup(values_hbm, idx_hbm, out_hbm, table_v, idx_v, out_v, *sems):
        in_sems, out_sems = sems[:_NBUF], sems[_NBUF:]
        wid = lax.axis_index("s") * _NC + lax.axis_index("c")
        base = wid * rows_per_w
        pltpu.sync_copy(values_hbm, table_v.at[pl.ds(0, n_values)])

        for b in range(_NBUF):
            pltpu.async_copy(
                idx_hbm.at[pl.ds(base + b * chunk_rows, chunk_rows), :],
                idx_v.at[b],
                in_sems[b],
            )

        @pl.loop(0, n_chunks // _NBUF)
        def _outer(o):
            for b in range(_NBUF):
                g = o * _NBUF + b
                row0 = base + g * chunk_rows
                rows = pl.ds(row0, chunk_rows)
                pltpu.make_async_copy(
                    idx_hbm.at[rows, :], idx_v.at[b], in_sems[b]
                ).wait()

                @pl.when(g >= _NBUF)
                def _():
                    pltpu.make_async_copy(
                        out_v.at[b], out_hbm.at[rows, :], out_sems[b]
                    ).wait()

                @plsc.parallel_loop(0, chunk_rows, unroll=2)
                def _row(r):
                    for k in col_offs:
                        iv = idx_v[b, r, pl.ds(k, _L)]
                        out_v[b, r, pl.ds(k, _L)] = plsc.load_gather(
                            table_v, [iv]
                        )

                pltpu.async_copy(out_v.at[b], out_hbm.at[rows, :], out_sems[b])

                @pl.when(g + _NBUF < n_chunks)
                def _():
                    nxt = pl.ds(base + (g + _NBUF) * chunk_rows, chunk_rows)
                    pltpu.async_copy(idx_hbm.at[nxt, :], idx_v.at[b], in_sems[b])

        for b in range(_NBUF):
            rows = pl.ds(base + (n_chunks - _NBUF + b) * chunk_rows, chunk_rows)
            pltpu.make_async_copy(
                out_v.at[b], out_hbm.at[rows, :], out_sems[b]
            ).wait()

    return lookup


def kernel(values, index):
    n_rows, n_cols = index.shape
    lookup = _make_lookup(n_rows, n_cols, values.shape[0], chunk_rows=64)
    return lookup(values, index)


# speedup vs baseline: 4.5525x; 1.6925x over previous
import functools

import jax
import jax.numpy as jnp
from jax import lax
from jax.experimental import pallas as pl
from jax.experimental.pallas import tpu as pltpu
from jax.experimental.pallas import tpu_sc as plsc

_INFO = plsc.get_sparse_core_info()
_NC, _NS, _L = _INFO.num_cores, _INFO.num_subcores, _INFO.num_lanes
_NW = _NC * _NS
_NBUF = 2


def _make_lookup(n_rows: int, n_cols: int, n_values: int, chunk_rows: int):
    cols_per_w = n_cols // _NW
    assert cols_per_w * _NW == n_cols
    n_chunks = n_rows // chunk_rows
    assert n_chunks * chunk_rows == n_rows
    assert cols_per_w % _L == 0
    mesh = plsc.VectorSubcoreMesh(core_axis_name="c", subcore_axis_name="s")
    n_pad = 16 * ((n_values + 15) // 16)

    @functools.partial(
        pl.kernel,
        mesh=mesh,
        out_type=jax.ShapeDtypeStruct((n_rows, n_cols), jnp.float32),
        scratch_types=[
            pltpu.VMEM((n_pad,), jnp.float32),
            pltpu.VMEM((_NBUF, chunk_rows, cols_per_w), jnp.int32),
            pltpu.VMEM((_NBUF, chunk_rows, cols_per_w), jnp.float32),
        ]
        + [pltpu.SemaphoreType.DMA] * (2 * _NBUF),
        compiler_params=pltpu.CompilerParams(needs_layout_passes=False),
    )
    def lookup(values_hbm, idx_hbm, out_hbm, table_v, idx_v, out_v, *sems):
        in_sems, out_sems = sems[:_NBUF], sems[_NBUF:]
        wid = lax.axis_index("s") * _NC + lax.axis_index("c")
        cols = pl.ds(wid * cols_per_w, cols_per_w)
        pltpu.sync_copy(values_hbm, table_v.at[pl.ds(0, n_values)])

        for b in range(min(_NBUF, n_chunks)):
            pltpu.async_copy(
                idx_hbm.at[pl.ds(b * chunk_rows, chunk_rows), cols],
                idx_v.at[b],
                in_sems[b],
            )

        for g in range(n_chunks):
            b = g % _NBUF
            rows = pl.ds(g * chunk_rows, chunk_rows)
            pltpu.make_async_copy(
                idx_hbm.at[rows, cols], idx_v.at[b], in_sems[b]
            ).wait()
            if g >= _NBUF:
                pltpu.make_async_copy(
                    out_v.at[b], out_hbm.at[rows, cols], out_sems[b]
                ).wait()

            @plsc.parallel_loop(0, chunk_rows)
            def _row(r):
                for k in range(cols_per_w // _L):
                    iv = idx_v[b, r, pl.ds(k * _L, _L)]
                    out_v[b, r, pl.ds(k * _L, _L)] = plsc.load_gather(
                        table_v, [iv]
                    )

            pltpu.async_copy(out_v.at[b], out_hbm.at[rows, cols], out_sems[b])
            if g + _NBUF < n_chunks:
                nxt = pl.ds((g + _NBUF) * chunk_rows, chunk_rows)
                pltpu.async_copy(idx_hbm.at[nxt, cols], idx_v.at[b], in_sems[b])

        for g in range(max(0, n_chunks - _NBUF), n_chunks):
            rows = pl.ds(g * chunk_rows, chunk_rows)
            pltpu.make_async_copy(
                out_v.at[g % _NBUF], out_hbm.at[rows, cols], out_sems[g % _NBUF]
            ).wait()

    return lookup


def kernel(values, index):
    n_rows, n_cols = index.shape
    idx_t = index.T
    lookup = _make_lookup(n_cols, n_rows, values.shape[0], chunk_rows=40)
    out_t = lookup(values, idx_t)
    return out_t.T


# speedup vs baseline: 4.9602x; 1.0896x over previous
import functools

import jax
import jax.numpy as jnp
from jax import lax
from jax.experimental import pallas as pl
from jax.experimental.pallas import tpu as pltpu
from jax.experimental.pallas import tpu_sc as plsc

_INFO = plsc.get_sparse_core_info()
_NC, _NS, _L = _INFO.num_cores, _INFO.num_subcores, _INFO.num_lanes
_NW = _NC * _NS
_NBUF = 2


def _make_lookup(n_rows: int, n_cols: int, n_values: int, chunk_cols: int):
    cols_per_w = n_cols // _NW
    assert cols_per_w * _NW == n_cols
    n_chunks = cols_per_w // chunk_cols
    assert n_chunks * chunk_cols == cols_per_w
    assert n_chunks % _NBUF == 0 and chunk_cols % _L == 0
    mesh = plsc.VectorSubcoreMesh(core_axis_name="c", subcore_axis_name="s")
    n_pad = 16 * ((n_values + 15) // 16)

    @functools.partial(
        pl.kernel,
        mesh=mesh,
        out_type=jax.ShapeDtypeStruct((n_rows, n_cols), jnp.float32),
        scratch_types=[
            pltpu.VMEM((n_pad,), jnp.float32),
            pltpu.VMEM((_NBUF, n_rows, chunk_cols), jnp.int32),
            pltpu.VMEM((_NBUF, n_rows, chunk_cols), jnp.float32),
        ]
        + [pltpu.SemaphoreType.DMA] * (2 * _NBUF),
        compiler_params=pltpu.CompilerParams(needs_layout_passes=False),
    )
    def lookup(values_hbm, idx_hbm, out_hbm, table_v, idx_v, out_v, *sems):
        in_sems, out_sems = sems[:_NBUF], sems[_NBUF:]
        wid = lax.axis_index("s") * _NC + lax.axis_index("c")
        col0 = wid * cols_per_w
        pltpu.sync_copy(values_hbm, table_v.at[pl.ds(0, n_values)])

        for b in range(_NBUF):
            pltpu.async_copy(
                idx_hbm.at[:, pl.ds(col0 + b * chunk_cols, chunk_cols)],
                idx_v.at[b],
                in_sems[b],
            )

        @pl.loop(0, n_chunks // _NBUF)
        def _outer(o):
            for b in range(_NBUF):
                g = o * _NBUF + b
                cols = pl.ds(col0 + g * chunk_cols, chunk_cols)
                pltpu.make_async_copy(
                    idx_hbm.at[:, cols], idx_v.at[b], in_sems[b]
                ).wait()

                @pl.when(g >= _NBUF)
                def _():
                    pltpu.make_async_copy(
                        out_v.at[b], out_hbm.at[:, cols], out_sems[b]
                    ).wait()

                @plsc.parallel_loop(0, n_rows)
                def _row(r):
                    for k in range(chunk_cols // _L):
                        iv = idx_v[b, r, pl.ds(k * _L, _L)]
                        out_v[b, r, pl.ds(k * _L, _L)] = plsc.load_gather(
                            table_v, [iv]
                        )

                pltpu.async_copy(out_v.at[b], out_hbm.at[:, cols], out_sems[b])

                @pl.when(g + _NBUF < n_chunks)
                def _():
                    nxt = pl.ds(col0 + (g + _NBUF) * chunk_cols, chunk_cols)
                    pltpu.async_copy(idx_hbm.at[:, nxt], idx_v.at[b], in_sems[b])

        for b in range(_NBUF):
            cols = pl.ds(col0 + (n_chunks - _NBUF + b) * chunk_cols, chunk_cols)
            pltpu.make_async_copy(
                out_v.at[b], out_hbm.at[:, cols], out_sems[b]
            ).wait()

    return lookup


def kernel(values, index):
    n_rows, n_cols = index.shape
    idx_t = index.T
    lookup = _make_lookup(n_cols, n_rows, values.shape[0], chunk_cols=128)
    out_t = lookup(values, idx_t)
    return out_t.T
